# Initial kernel scaffold; baseline (speedup 1.0000x reference)
import jax
import jax.numpy as jnp
from jax.experimental import pallas as pl


def kernel(x_nchw, expand_w, expand_b):
    raise NotImplementedError("write your pallas kernel here")


# speedup vs baseline: 1.1783x; 1.1783x over previous
import jax
import jax.numpy as jnp
from jax.experimental import pallas as pl
from jax.experimental.pallas import tpu as pltpu


def _fused_kernel(h2, w2, x_ref, wt_ref, b_ref, o_ref, t_scr):
    x = x_ref[0]
    t = jnp.transpose(x)
    t_scr[...] = t.reshape(2 * h2, 2 * w2, t.shape[-1])
    ev = pl.ds(0, h2, 2)
    od = pl.ds(1, h2, 2)
    evw = pl.ds(0, w2, 2)
    odw = pl.ds(1, w2, 2)
    p3 = (t_scr[ev, evw, :] + t_scr[ev, odw, :]
          + t_scr[od, evw, :] + t_scr[od, odw, :])
    p = p3.reshape(h2 * w2, p3.shape[-1])
    y = jnp.dot(p, wt_ref[...], preferred_element_type=jnp.float32)
    y = y + b_ref[...]
    o_ref[0] = jnp.transpose(y)


def kernel(x_nchw, expand_w, expand_b):
    B, C, H, W = x_nchw.shape
    C_out = expand_w.shape[0]
    H2, W2 = H // 2, W // 2
    if (H % 2) or (W % 2):
        x_nchw = x_nchw[:, :, : 2 * H2, : 2 * W2]

    xf = x_nchw.reshape(B, C, 2 * H2 * 2 * W2)
    wt = (jnp.transpose(expand_w) * 0.25).astype(x_nchw.dtype)
    b2 = jnp.asarray(expand_b, jnp.float32).reshape(1, C_out)

    out_flat = pl.pallas_call(
        lambda x_ref, wt_ref, b_ref, o_ref, t_scr: _fused_kernel(
            H2, W2, x_ref, wt_ref, b_ref, o_ref, t_scr),
        out_shape=jax.ShapeDtypeStruct((B, C_out, H2 * W2), x_nchw.dtype),
        grid=(B,),
        in_specs=[
            pl.BlockSpec((1, C, 2 * H2 * 2 * W2), lambda i: (i, 0, 0)),
            pl.BlockSpec((C, C_out), lambda i: (0, 0)),
            pl.BlockSpec((1, C_out), lambda i: (0, 0)),
        ],
        out_specs=pl.BlockSpec((1, C_out, H2 * W2), lambda i: (i, 0, 0)),
        scratch_shapes=[pltpu.VMEM((2 * H2, 2 * W2, C), jnp.float32)],
        compiler_params=pltpu.CompilerParams(
            dimension_semantics=("parallel",),
            vmem_limit_bytes=64 * 1024 * 1024,
        ),
    )(xf, wt, b2)

    return out_flat.reshape(B, C_out, H2, W2)
